# baseline (speedup 1.0000x reference)
import functools

import jax
import jax.numpy as jnp
from jax import lax
from jax.experimental import pallas as pl
from jax.experimental.pallas import tpu as pltpu
from jax.experimental.pallas import tpu_sc as plsc

B = 16384
EMBED = 64
PAIR = 2 * EMBED

NC, NS = 2, 16
NW = NC * NS
B_PER_W = B // NW


@functools.cache
def _make_sc_gather():
    mesh = plsc.VectorSubcoreMesh(
        core_axis_name="c", subcore_axis_name="s",
        num_cores=NC, num_subcores=NS)

    @functools.partial(
        pl.kernel,
        mesh=mesh,
        out_type=jax.ShapeDtypeStruct((B, 2 * PAIR), jnp.float32),
        scratch_types=[
            pltpu.VMEM((B_PER_W,), jnp.int32),
            pltpu.VMEM((B_PER_W,), jnp.int32),
            pltpu.VMEM((B_PER_W // 2, 2 * PAIR), jnp.float32),
            pltpu.SemaphoreType.DMA,
        ],
    )
    def _sc_gather(user_idx, item_idx, user_tab, item_tab,
                   out, uidx_v, iidx_v, rowsbuf, sem):
        wid = lax.axis_index("s") * NC + lax.axis_index("c")
        base = wid * B_PER_W
        pltpu.sync_copy(user_idx.at[pl.ds(base, B_PER_W)], uidx_v)
        pltpu.sync_copy(item_idx.at[pl.ds(base, B_PER_W)], iidx_v)

        half = B_PER_W // 2
        for h in range(2):
            def body(j, _, h=h):
                k0 = h * half + j * 16
                uv = uidx_v[pl.ds(k0, 16)]
                iv = iidx_v[pl.ds(k0, 16)]
                for l in range(16):
                    pltpu.async_copy(
                        user_tab.at[uv[l]],
                        rowsbuf.at[j * 16 + l, pl.ds(0, PAIR)], sem)
                    pltpu.async_copy(
                        item_tab.at[iv[l]],
                        rowsbuf.at[j * 16 + l, pl.ds(PAIR, PAIR)], sem)
                return 0

            lax.fori_loop(0, half // 16, body, 0)
            pltpu.make_async_copy(out.at[pl.ds(0, half)], rowsbuf, sem).wait()
            pltpu.sync_copy(rowsbuf, out.at[pl.ds(base + h * half, half)])

    return _sc_gather


def _mlp_body(x_ref, up_ref, ip_ref, w1u_ref, w1v_ref, b1_ref, w2_ref,
              b2_ref, w3_ref, b3_ref, out_ref):
    x = x_ref[...]
    u = jnp.where(up_ref[...] > 0, x[:, EMBED:PAIR], x[:, 0:EMBED])
    v = jnp.where(ip_ref[...] > 0, x[:, PAIR + EMBED:], x[:, PAIR:PAIR + EMBED])
    h1 = jnp.dot(u, w1u_ref[...], preferred_element_type=jnp.float32)
    h1 += jnp.dot(v, w1v_ref[...], preferred_element_type=jnp.float32)
    h1 = jnp.maximum(h1 + b1_ref[...], 0.0)
    h2 = jnp.dot(h1, w2_ref[...], preferred_element_type=jnp.float32)
    h2 = jnp.maximum(h2 + b2_ref[...], 0.0)
    logit = jnp.sum(h2 * w3_ref[...], axis=1, keepdims=True) + b3_ref[...]
    out_ref[...] = 5.0 / (1.0 + jnp.exp(-logit))


def _tc_mlp(x, upar, ipar, W1, b1, W2, b2, W3, b3):
    blk = 2048
    grid = (B // blk,)
    full = lambda shape: pl.BlockSpec(shape, lambda i: (0, 0))
    return pl.pallas_call(
        _mlp_body,
        grid=grid,
        in_specs=[
            pl.BlockSpec((blk, 2 * PAIR), lambda i: (i, 0)),
            pl.BlockSpec((blk, 1), lambda i: (i, 0)),
            pl.BlockSpec((blk, 1), lambda i: (i, 0)),
            full((EMBED, 128)),
            full((EMBED, 128)),
            full((1, 128)),
            full((128, 64)),
            full((1, 64)),
            full((1, 64)),
            full((1, 1)),
        ],
        out_specs=pl.BlockSpec((blk, 1), lambda i: (i, 0)),
        out_shape=jax.ShapeDtypeStruct((B, 1), jnp.float32),
    )(x, upar, ipar, W1[:EMBED], W1[EMBED:], b1.reshape(1, -1),
      W2, b2.reshape(1, -1), W3.reshape(1, -1), b3.reshape(1, 1))


def kernel(user_input, item_input, user_table, item_table, W1, b1, W2, b2, W3, b3):
    n_users = user_table.shape[0]
    n_items = item_table.shape[0]
    user_pairs = user_table.reshape(n_users // 2, PAIR)
    item_pairs = item_table.reshape(n_items // 2, PAIR)
    x = _make_sc_gather()(
        user_input >> 1, item_input >> 1, user_pairs, item_pairs)
    upar = (user_input & 1).astype(jnp.int32).reshape(B, 1)
    ipar = (item_input & 1).astype(jnp.int32).reshape(B, 1)
    return _tc_mlp(x, upar, ipar, W1, b1, W2, b2, W3, b3)
